# baseline (speedup 1.0000x reference)
import functools

import jax
import jax.numpy as jnp
from jax import lax
from jax.experimental import pallas as pl
from jax.experimental.pallas import tpu as pltpu
from jax.experimental.pallas import tpu_sc as plsc

B = 4096
L = 50
D = 32
NR5 = 5
EPS_BN = 1e-5

NW = 32
RTOT = B * L
RPW = RTOT // NW
CH = 128
NCH = RPW // CH
BPW = B // NW

PK = 4
BP = B // PK
DP = PK * D


def _sc_gather(hist_idx, nodes_idx, i2e_w, u2e_w):
    mesh = plsc.VectorSubcoreMesh(core_axis_name="c", subcore_axis_name="s")

    @functools.partial(
        pl.kernel,
        mesh=mesh,
        compiler_params=pltpu.CompilerParams(use_tc_tiling_on_sc=False),
        out_type=(
            jax.ShapeDtypeStruct((RTOT, D), jnp.float32),
            jax.ShapeDtypeStruct((B, D), jnp.float32),
        ),
        scratch_types=[
            pltpu.VMEM((NCH, CH), jnp.int32),
            pltpu.VMEM((CH, D), jnp.float32),
            pltpu.VMEM((CH, D), jnp.float32),
            pltpu.VMEM((1, CH), jnp.int32),
            pltpu.VMEM((CH, D), jnp.float32),
            pltpu.SemaphoreType.DMA,
        ],
    )
    def k(idx_hbm, nodes_hbm, i2e_hbm, u2e_hbm, eui_out, self_out,
          idxv, buf0, buf1, idxu, bufu, sem):
        c = lax.axis_index("c")
        s = lax.axis_index("s")
        wid = s * 2 + c
        pltpu.sync_copy(idx_hbm.at[wid], idxv)

        def body(j, _):
            base = pl.multiple_of(wid * RPW + j * CH, CH)
            pltpu.async_copy(i2e_hbm.at[idxv.at[j]], buf0, sem).wait()
            pltpu.sync_copy(buf0, eui_out.at[pl.ds(base, CH)])
            return 0

        lax.fori_loop(0, NCH, body, 0, unroll=False)

        pltpu.sync_copy(nodes_hbm.at[wid], idxu)
        pltpu.async_copy(u2e_hbm.at[idxu.at[0]], bufu, sem).wait()
        sbase = pl.multiple_of(wid * BPW, BPW)
        pltpu.sync_copy(bufu, self_out.at[pl.ds(sbase, BPW)])

    return k(hist_idx.reshape(NW, NCH, CH), nodes_idx.reshape(NW, 1, BPW),
             i2e_w, u2e_w)


TK = 4096


def _tc_linearize(table):
    n = table.shape[0]
    nblk = -(-n // TK)

    def kern(x_ref, o_ref):
        t = jnp.transpose(x_ref[...].reshape(D, TK // PK, PK), (1, 2, 0))
        o_ref[...] = t.reshape(TK // PK, DP)

    return pl.pallas_call(
        kern,
        grid=(nblk,),
        in_specs=[pl.BlockSpec((D, TK), lambda i: (0, i))],
        out_specs=pl.BlockSpec((TK // PK, DP), lambda i: (i, 0)),
        out_shape=jax.ShapeDtypeStruct((n // PK, DP), jnp.float32),
    )(table.T)


NBP = 256


def _tc_stage1(eui_p, oh_p, w20, g0b, g1b, g2b, bd1, gbp):
    def kern(eui_ref, oh_ref, w20_ref, g0_ref, g1_ref, g2_ref, bd1_ref,
             gb_ref, out_ref):
        w20v = w20_ref[...]
        g0 = g0_ref[...]
        g1 = g1_ref[...]
        g2 = g2_ref[...]
        bd1 = bd1_ref[...]
        gb = gb_ref[...]
        acc = jnp.zeros((NBP, DP), jnp.float32)
        for j in range(L):
            e = eui_ref[j]
            r = lax.dot_general(oh_ref[j], w20v, (((0,), (0,)), ((), ())),
                                preferred_element_type=jnp.float32)
            x = e * r
            z = (jnp.dot(e, g0, preferred_element_type=jnp.float32)
                 + jnp.dot(r, g1, preferred_element_type=jnp.float32)
                 + jnp.dot(x, g2, preferred_element_type=jnp.float32)
                 + gb)
            alpha = jax.nn.sigmoid(z)
            o = alpha * e + (1.0 - alpha) * r
            n2 = jnp.dot(o * o, bd1, preferred_element_type=jnp.float32)
            n = jnp.sqrt(n2)
            acc = acc + o / jnp.maximum(n, 1e-12)
        out_ref[...] = acc

    return pl.pallas_call(
        kern,
        grid=(BP // NBP,),
        in_specs=[
            pl.BlockSpec((L, NBP, DP), lambda b: (0, b, 0)),
            pl.BlockSpec((L, PK * NR5, NBP), lambda b: (0, 0, b)),
            pl.BlockSpec((PK * NR5, DP), lambda b: (0, 0)),
            pl.BlockSpec((DP, DP), lambda b: (0, 0)),
            pl.BlockSpec((DP, DP), lambda b: (0, 0)),
            pl.BlockSpec((DP, DP), lambda b: (0, 0)),
            pl.BlockSpec((DP, DP), lambda b: (0, 0)),
            pl.BlockSpec((1, DP), lambda b: (0, 0)),
        ],
        out_specs=pl.BlockSpec((NBP, DP), lambda b: (b, 0)),
        out_shape=jax.ShapeDtypeStruct((BP, DP), jnp.float32),
    )(eui_p, oh_p, w20, g0b, g1b, g2b, bd1, gbp)


def _selu(x):
    a = 1.6732632423543772848170429916717
    s = 1.0507009873554804934193349852946
    return s * jnp.where(x > 0, x, a * (jnp.exp(x) - 1.0))


def _tc_stage2(embed_p, sf_p, inwb, inbp, outwb, outbp, g10, g11, g12, g1bp,
               bngp, bnbp, bn1gp, bn1bp, mavg):
    def kern(em_ref, sf_ref, inw_ref, inb_ref, outw_ref, outb_ref,
             g10_ref, g11_ref, g12_ref, g1b_ref, bng_ref, bnb_ref,
             bn1g_ref, bn1b_ref, mavg_ref, out_ref):
        em = em_ref[...]
        mv = mavg_ref[...]
        m = jnp.dot(jnp.mean(em, axis=0, keepdims=True), mv,
                    preferred_element_type=jnp.float32)
        v = jnp.dot(jnp.mean((em - m) ** 2, axis=0, keepdims=True), mv,
                    preferred_element_type=jnp.float32)
        xb = (em - m) / jnp.sqrt(v + EPS_BN) * bng_ref[...] + bnb_ref[...]
        xb = _selu(jnp.dot(xb, inw_ref[...],
                           preferred_element_type=jnp.float32) + inb_ref[...])
        m1 = jnp.dot(jnp.mean(xb, axis=0, keepdims=True), mv,
                     preferred_element_type=jnp.float32)
        v1 = jnp.dot(jnp.mean((xb - m1) ** 2, axis=0, keepdims=True), mv,
                     preferred_element_type=jnp.float32)
        xb = (xb - m1) / jnp.sqrt(v1 + EPS_BN) * bn1g_ref[...] + bn1b_ref[...]
        neigh = jnp.dot(xb, outw_ref[...],
                        preferred_element_type=jnp.float32) + outb_ref[...]
        sfv = sf_ref[...]
        z = (jnp.dot(sfv, g10_ref[...], preferred_element_type=jnp.float32)
             + jnp.dot(neigh, g11_ref[...],
                       preferred_element_type=jnp.float32)
             + jnp.dot(sfv * neigh, g12_ref[...],
                       preferred_element_type=jnp.float32)
             + g1b_ref[...])
        beta = jax.nn.sigmoid(z)
        out_ref[...] = beta * sfv + (1.0 - beta) * neigh

    return pl.pallas_call(
        kern,
        out_shape=jax.ShapeDtypeStruct((BP, DP), jnp.float32),
    )(embed_p, sf_p, inwb, inbp, outwb, outbp, g10, g11, g12, g1bp,
      bngp, bnbp, bn1gp, bn1bp, mavg)


def _bd(w):
    return jnp.kron(jnp.eye(PK, dtype=jnp.float32), w)


def _tile_row(v):
    return jnp.tile(v, PK).reshape(1, DP)


def kernel(nodes, history_ui, history_r, u2e_w, i2e_w, r2e_w, l1W, l1b,
           a1W, a1b, a2W, a2b, a3W, a3b, gate_W, gate_b, gate1_W, gate1_b,
           bn_g, bn_b, inW, inb, bn1_g, bn1_b, outW, outb):
    hist_idx = history_ui.astype(jnp.int32).T.reshape(-1)
    nodes_idx = nodes.astype(jnp.int32)

    i2e_lin = _tc_linearize(i2e_w).reshape(i2e_w.shape)
    u2e_lin = _tc_linearize(u2e_w).reshape(u2e_w.shape)

    eui_flat, sf = _sc_gather(hist_idx, nodes_idx, i2e_lin, u2e_lin)
    eui_p = eui_flat.reshape(L, BP, DP)
    sf_p = sf.reshape(BP, DP)

    hr3 = history_r.astype(jnp.int32).T.reshape(L, BP, PK)
    hr3 = jnp.transpose(hr3, (0, 2, 1))
    kk = jnp.arange(PK * NR5, dtype=jnp.int32)
    oh_p = (hr3[:, kk // NR5, :] == (kk % NR5)[None, :, None]
            ).astype(jnp.float32)

    gate_wt = gate_W.T
    embed_p = _tc_stage1(
        eui_p, oh_p,
        _bd(r2e_w),
        _bd(gate_wt[:D]), _bd(gate_wt[D:2 * D]), _bd(gate_wt[2 * D:]),
        jnp.kron(jnp.eye(PK, dtype=jnp.float32),
                 jnp.ones((D, D), jnp.float32)),
        _tile_row(gate_b))

    g1t = gate1_W.T
    mavg = jnp.kron(jnp.full((PK, PK), 1.0 / PK, jnp.float32),
                    jnp.eye(D, dtype=jnp.float32))
    out_p = _tc_stage2(
        embed_p, sf_p,
        _bd(inW.T), _tile_row(inb), _bd(outW.T), _tile_row(outb),
        _bd(g1t[:D]), _bd(g1t[D:2 * D]), _bd(g1t[2 * D:]), _tile_row(gate1_b),
        _tile_row(bn_g), _tile_row(bn_b), _tile_row(bn1_g), _tile_row(bn1_b),
        mavg)

    return out_p.reshape(B, D)


# speedup vs baseline: 5.9289x; 5.9289x over previous
import functools

import jax
import jax.numpy as jnp
from jax import lax
from jax.experimental import pallas as pl
from jax.experimental.pallas import tpu as pltpu
from jax.experimental.pallas import tpu_sc as plsc

B = 4096
L = 50
D = 32
NR5 = 5
EPS_BN = 1e-5

NW = 32
RTOT = B * L
RPW = RTOT // NW
CH = 128
NCH = RPW // CH
BPW = B // NW

PK = 4
BP = B // PK
DP = PK * D


def _sc_gather(hist_idx, nodes_idx, i2e_w, u2e_w):
    mesh = plsc.VectorSubcoreMesh(core_axis_name="c", subcore_axis_name="s")

    @functools.partial(
        pl.kernel,
        mesh=mesh,
        compiler_params=pltpu.CompilerParams(use_tc_tiling_on_sc=False),
        out_type=(
            jax.ShapeDtypeStruct((RTOT, D), jnp.float32),
            jax.ShapeDtypeStruct((B, D), jnp.float32),
        ),
        scratch_types=[
            pltpu.VMEM((NCH, CH), jnp.int32),
            pltpu.VMEM((CH, D), jnp.float32),
            pltpu.VMEM((CH, D), jnp.float32),
            pltpu.VMEM((1, CH), jnp.int32),
            pltpu.VMEM((CH, D), jnp.float32),
            pltpu.SemaphoreType.DMA,
        ],
    )
    def k(idx_hbm, nodes_hbm, i2e_hbm, u2e_hbm, eui_out, self_out,
          idxv, buf0, buf1, idxu, bufu, sem):
        c = lax.axis_index("c")
        s = lax.axis_index("s")
        wid = s * 2 + c
        pltpu.sync_copy(idx_hbm.at[wid], idxv)

        def body(j, _):
            base = pl.multiple_of(wid * RPW + j * CH, CH)
            pltpu.async_copy(i2e_hbm.at[idxv.at[j]], buf0, sem).wait()
            pltpu.sync_copy(buf0, eui_out.at[pl.ds(base, CH)])
            return 0

        lax.fori_loop(0, NCH, body, 0, unroll=False)

        pltpu.sync_copy(nodes_hbm.at[wid], idxu)
        pltpu.async_copy(u2e_hbm.at[idxu.at[0]], bufu, sem).wait()
        sbase = pl.multiple_of(wid * BPW, BPW)
        pltpu.sync_copy(bufu, self_out.at[pl.ds(sbase, BPW)])

    return k(hist_idx.reshape(NW, NCH, CH), nodes_idx.reshape(NW, 1, BPW),
             i2e_w, u2e_w)


TK = 4096


def _tc_linearize(table):
    n = table.shape[0]
    nblk = -(-n // TK)

    def kern(x_ref, o_ref, scr):
        scr[...] = jnp.transpose(x_ref[...])
        for q in range(PK):
            o_ref[:, q * D:(q + 1) * D] = scr[pl.Slice(q, TK // PK, PK), :]

    return pl.pallas_call(
        kern,
        grid=(nblk,),
        in_specs=[pl.BlockSpec((D, TK), lambda i: (0, i))],
        out_specs=pl.BlockSpec((TK // PK, DP), lambda i: (i, 0)),
        out_shape=jax.ShapeDtypeStruct((n // PK, DP), jnp.float32),
        scratch_shapes=[pltpu.VMEM((TK, D), jnp.float32)],
    )(table.T)


NBP = 256


def _tc_stage1(eui_p, oh_p, w20, g0b, g1b, g2b, bd1, gbp):
    def kern(eui_ref, oh_ref, w20_ref, g0_ref, g1_ref, g2_ref, bd1_ref,
             gb_ref, out_ref):
        w20v = w20_ref[...]
        g0 = g0_ref[...]
        g1 = g1_ref[...]
        g2 = g2_ref[...]
        bd1 = bd1_ref[...]
        gb = gb_ref[...]
        acc = jnp.zeros((NBP, DP), jnp.float32)
        for j in range(L):
            e = eui_ref[j]
            r = lax.dot_general(oh_ref[j], w20v, (((0,), (0,)), ((), ())),
                                preferred_element_type=jnp.float32)
            x = e * r
            z = (jnp.dot(e, g0, preferred_element_type=jnp.float32)
                 + jnp.dot(r, g1, preferred_element_type=jnp.float32)
                 + jnp.dot(x, g2, preferred_element_type=jnp.float32)
                 + gb)
            alpha = jax.nn.sigmoid(z)
            o = alpha * e + (1.0 - alpha) * r
            n2 = jnp.dot(o * o, bd1, preferred_element_type=jnp.float32)
            n = jnp.sqrt(n2)
            acc = acc + o / jnp.maximum(n, 1e-12)
        out_ref[...] = acc

    return pl.pallas_call(
        kern,
        grid=(BP // NBP,),
        in_specs=[
            pl.BlockSpec((L, NBP, DP), lambda b: (0, b, 0)),
            pl.BlockSpec((L, PK * NR5, NBP), lambda b: (0, 0, b)),
            pl.BlockSpec((PK * NR5, DP), lambda b: (0, 0)),
            pl.BlockSpec((DP, DP), lambda b: (0, 0)),
            pl.BlockSpec((DP, DP), lambda b: (0, 0)),
            pl.BlockSpec((DP, DP), lambda b: (0, 0)),
            pl.BlockSpec((DP, DP), lambda b: (0, 0)),
            pl.BlockSpec((1, DP), lambda b: (0, 0)),
        ],
        out_specs=pl.BlockSpec((NBP, DP), lambda b: (b, 0)),
        out_shape=jax.ShapeDtypeStruct((BP, DP), jnp.float32),
    )(eui_p, oh_p, w20, g0b, g1b, g2b, bd1, gbp)


def _selu(x):
    a = 1.6732632423543772848170429916717
    s = 1.0507009873554804934193349852946
    return s * jnp.where(x > 0, x, a * (jnp.exp(x) - 1.0))


def _tc_stage2(embed_p, sf_p, inwb, inbp, outwb, outbp, g10, g11, g12, g1bp,
               bngp, bnbp, bn1gp, bn1bp, mavg):
    def kern(em_ref, sf_ref, inw_ref, inb_ref, outw_ref, outb_ref,
             g10_ref, g11_ref, g12_ref, g1b_ref, bng_ref, bnb_ref,
             bn1g_ref, bn1b_ref, mavg_ref, out_ref):
        em = em_ref[...]
        mv = mavg_ref[...]
        m = jnp.dot(jnp.mean(em, axis=0, keepdims=True), mv,
                    preferred_element_type=jnp.float32)
        v = jnp.dot(jnp.mean((em - m) ** 2, axis=0, keepdims=True), mv,
                    preferred_element_type=jnp.float32)
        xb = (em - m) / jnp.sqrt(v + EPS_BN) * bng_ref[...] + bnb_ref[...]
        xb = _selu(jnp.dot(xb, inw_ref[...],
                           preferred_element_type=jnp.float32) + inb_ref[...])
        m1 = jnp.dot(jnp.mean(xb, axis=0, keepdims=True), mv,
                     preferred_element_type=jnp.float32)
        v1 = jnp.dot(jnp.mean((xb - m1) ** 2, axis=0, keepdims=True), mv,
                     preferred_element_type=jnp.float32)
        xb = (xb - m1) / jnp.sqrt(v1 + EPS_BN) * bn1g_ref[...] + bn1b_ref[...]
        neigh = jnp.dot(xb, outw_ref[...],
                        preferred_element_type=jnp.float32) + outb_ref[...]
        sfv = sf_ref[...]
        z = (jnp.dot(sfv, g10_ref[...], preferred_element_type=jnp.float32)
             + jnp.dot(neigh, g11_ref[...],
                       preferred_element_type=jnp.float32)
             + jnp.dot(sfv * neigh, g12_ref[...],
                       preferred_element_type=jnp.float32)
             + g1b_ref[...])
        beta = jax.nn.sigmoid(z)
        out_ref[...] = beta * sfv + (1.0 - beta) * neigh

    return pl.pallas_call(
        kern,
        out_shape=jax.ShapeDtypeStruct((BP, DP), jnp.float32),
    )(embed_p, sf_p, inwb, inbp, outwb, outbp, g10, g11, g12, g1bp,
      bngp, bnbp, bn1gp, bn1bp, mavg)


def _bd(w):
    return jnp.kron(jnp.eye(PK, dtype=jnp.float32), w)


def _tile_row(v):
    return jnp.tile(v, PK).reshape(1, DP)


def kernel(nodes, history_ui, history_r, u2e_w, i2e_w, r2e_w, l1W, l1b,
           a1W, a1b, a2W, a2b, a3W, a3b, gate_W, gate_b, gate1_W, gate1_b,
           bn_g, bn_b, inW, inb, bn1_g, bn1_b, outW, outb):
    hist_idx = history_ui.astype(jnp.int32).T.reshape(-1)
    nodes_idx = nodes.astype(jnp.int32)

    i2e_lin = _tc_linearize(i2e_w).reshape(i2e_w.shape)
    u2e_lin = _tc_linearize(u2e_w).reshape(u2e_w.shape)

    eui_flat, sf = _sc_gather(hist_idx, nodes_idx, i2e_lin, u2e_lin)
    eui_p = eui_flat.reshape(L, BP, DP)
    sf_p = sf.reshape(BP, DP)

    hr3 = history_r.astype(jnp.int32).T.reshape(L, BP, PK)
    hr3 = jnp.transpose(hr3, (0, 2, 1))
    kk = jnp.arange(PK * NR5, dtype=jnp.int32)
    oh_p = (hr3[:, kk // NR5, :] == (kk % NR5)[None, :, None]
            ).astype(jnp.float32)

    gate_wt = gate_W.T
    embed_p = _tc_stage1(
        eui_p, oh_p,
        _bd(r2e_w),
        _bd(gate_wt[:D]), _bd(gate_wt[D:2 * D]), _bd(gate_wt[2 * D:]),
        jnp.kron(jnp.eye(PK, dtype=jnp.float32),
                 jnp.ones((D, D), jnp.float32)),
        _tile_row(gate_b))

    g1t = gate1_W.T
    mavg = jnp.kron(jnp.full((PK, PK), 1.0 / PK, jnp.float32),
                    jnp.eye(D, dtype=jnp.float32))
    out_p = _tc_stage2(
        embed_p, sf_p,
        _bd(inW.T), _tile_row(inb), _bd(outW.T), _tile_row(outb),
        _bd(g1t[:D]), _bd(g1t[D:2 * D]), _bd(g1t[2 * D:]), _tile_row(gate1_b),
        _tile_row(bn_g), _tile_row(bn_b), _tile_row(bn1_g), _tile_row(bn1_b),
        mavg)

    return out_p.reshape(B, D)


# speedup vs baseline: 5.9343x; 1.0009x over previous
import functools

import jax
import jax.numpy as jnp
from jax import lax
from jax.experimental import pallas as pl
from jax.experimental.pallas import tpu as pltpu
from jax.experimental.pallas import tpu_sc as plsc

B = 4096
L = 50
D = 32
NR5 = 5
EPS_BN = 1e-5

NW = 32
RTOT = B * L
RPW = RTOT // NW
CH = 128
NCH = RPW // CH
BPW = B // NW

PK = 4
BP = B // PK
DP = PK * D


def _sc_gather(hist_idx, nodes_idx, i2e_w, u2e_w):
    mesh = plsc.VectorSubcoreMesh(core_axis_name="c", subcore_axis_name="s")

    @functools.partial(
        pl.kernel,
        mesh=mesh,
        compiler_params=pltpu.CompilerParams(use_tc_tiling_on_sc=False),
        out_type=(
            jax.ShapeDtypeStruct((RTOT, D), jnp.float32),
            jax.ShapeDtypeStruct((B, D), jnp.float32),
        ),
        scratch_types=[
            pltpu.VMEM((NCH, CH), jnp.int32),
            pltpu.VMEM((CH, D), jnp.float32),
            pltpu.VMEM((CH, D), jnp.float32),
            pltpu.VMEM((1, CH), jnp.int32),
            pltpu.VMEM((CH, D), jnp.float32),
            pltpu.SemaphoreType.DMA,
        ],
    )
    def k(idx_hbm, nodes_hbm, i2e_hbm, u2e_hbm, eui_out, self_out,
          idxv, buf0, buf1, idxu, bufu, sem):
        c = lax.axis_index("c")
        s = lax.axis_index("s")
        wid = s * 2 + c
        pltpu.sync_copy(idx_hbm.at[wid], idxv)

        def body(j2, _):
            j = j2 * 2
            base0 = pl.multiple_of(wid * RPW + j * CH, CH)
            base1 = pl.multiple_of(wid * RPW + (j + 1) * CH, CH)
            h0 = pltpu.async_copy(i2e_hbm.at[idxv.at[j]], buf0, sem)
            h1 = pltpu.async_copy(i2e_hbm.at[idxv.at[j + 1]], buf1, sem)
            h0.wait()
            pltpu.sync_copy(buf0, eui_out.at[pl.ds(base0, CH)])
            h1.wait()
            pltpu.sync_copy(buf1, eui_out.at[pl.ds(base1, CH)])
            return 0

        lax.fori_loop(0, NCH // 2, body, 0, unroll=False)

        pltpu.sync_copy(nodes_hbm.at[wid], idxu)
        pltpu.async_copy(u2e_hbm.at[idxu.at[0]], bufu, sem).wait()
        sbase = pl.multiple_of(wid * BPW, BPW)
        pltpu.sync_copy(bufu, self_out.at[pl.ds(sbase, BPW)])

    return k(hist_idx.reshape(NW, NCH, CH), nodes_idx.reshape(NW, 1, BPW),
             i2e_w, u2e_w)


TK = 4096


def _tc_linearize(table):
    n = table.shape[0]
    nblk = -(-n // TK)

    def kern(x_ref, o_ref, scr):
        scr[...] = jnp.transpose(x_ref[...])
        for q in range(PK):
            o_ref[:, q * D:(q + 1) * D] = scr[pl.Slice(q, TK // PK, PK), :]

    return pl.pallas_call(
        kern,
        grid=(nblk,),
        in_specs=[pl.BlockSpec((D, TK), lambda i: (0, i))],
        out_specs=pl.BlockSpec((TK // PK, DP), lambda i: (i, 0)),
        out_shape=jax.ShapeDtypeStruct((n // PK, DP), jnp.float32),
        scratch_shapes=[pltpu.VMEM((TK, D), jnp.float32)],
    )(table.T)


NBP = 256


def _tc_stage1(eui_p, oh_p, w20, g0b, g1b, g2b, bd1, gbp):
    def kern(eui_ref, oh_ref, w20_ref, g0_ref, g1_ref, g2_ref, bd1_ref,
             gb_ref, out_ref):
        w20v = w20_ref[...]
        g0 = g0_ref[...]
        g1 = g1_ref[...]
        g2 = g2_ref[...]
        bd1 = bd1_ref[...]
        gb = gb_ref[...]
        acc = jnp.zeros((NBP, DP), jnp.float32)
        for j in range(L):
            e = eui_ref[j]
            r = lax.dot_general(oh_ref[j], w20v, (((0,), (0,)), ((), ())),
                                preferred_element_type=jnp.float32)
            x = e * r
            z = (jnp.dot(e, g0, preferred_element_type=jnp.float32)
                 + jnp.dot(r, g1, preferred_element_type=jnp.float32)
                 + jnp.dot(x, g2, preferred_element_type=jnp.float32)
                 + gb)
            alpha = jax.nn.sigmoid(z)
            o = alpha * e + (1.0 - alpha) * r
            n2 = jnp.dot(o * o, bd1, preferred_element_type=jnp.float32)
            n = jnp.sqrt(n2)
            acc = acc + o / jnp.maximum(n, 1e-12)
        out_ref[...] = acc

    return pl.pallas_call(
        kern,
        grid=(BP // NBP,),
        in_specs=[
            pl.BlockSpec((L, NBP, DP), lambda b: (0, b, 0)),
            pl.BlockSpec((L, PK * NR5, NBP), lambda b: (0, 0, b)),
            pl.BlockSpec((PK * NR5, DP), lambda b: (0, 0)),
            pl.BlockSpec((DP, DP), lambda b: (0, 0)),
            pl.BlockSpec((DP, DP), lambda b: (0, 0)),
            pl.BlockSpec((DP, DP), lambda b: (0, 0)),
            pl.BlockSpec((DP, DP), lambda b: (0, 0)),
            pl.BlockSpec((1, DP), lambda b: (0, 0)),
        ],
        out_specs=pl.BlockSpec((NBP, DP), lambda b: (b, 0)),
        out_shape=jax.ShapeDtypeStruct((BP, DP), jnp.float32),
    )(eui_p, oh_p, w20, g0b, g1b, g2b, bd1, gbp)


def _selu(x):
    a = 1.6732632423543772848170429916717
    s = 1.0507009873554804934193349852946
    return s * jnp.where(x > 0, x, a * (jnp.exp(x) - 1.0))


def _tc_stage2(embed_p, sf_p, inwb, inbp, outwb, outbp, g10, g11, g12, g1bp,
               bngp, bnbp, bn1gp, bn1bp, mavg):
    def kern(em_ref, sf_ref, inw_ref, inb_ref, outw_ref, outb_ref,
             g10_ref, g11_ref, g12_ref, g1b_ref, bng_ref, bnb_ref,
             bn1g_ref, bn1b_ref, mavg_ref, out_ref):
        em = em_ref[...]
        mv = mavg_ref[...]
        m = jnp.dot(jnp.mean(em, axis=0, keepdims=True), mv,
                    preferred_element_type=jnp.float32)
        v = jnp.dot(jnp.mean((em - m) ** 2, axis=0, keepdims=True), mv,
                    preferred_element_type=jnp.float32)
        xb = (em - m) / jnp.sqrt(v + EPS_BN) * bng_ref[...] + bnb_ref[...]
        xb = _selu(jnp.dot(xb, inw_ref[...],
                           preferred_element_type=jnp.float32) + inb_ref[...])
        m1 = jnp.dot(jnp.mean(xb, axis=0, keepdims=True), mv,
                     preferred_element_type=jnp.float32)
        v1 = jnp.dot(jnp.mean((xb - m1) ** 2, axis=0, keepdims=True), mv,
                     preferred_element_type=jnp.float32)
        xb = (xb - m1) / jnp.sqrt(v1 + EPS_BN) * bn1g_ref[...] + bn1b_ref[...]
        neigh = jnp.dot(xb, outw_ref[...],
                        preferred_element_type=jnp.float32) + outb_ref[...]
        sfv = sf_ref[...]
        z = (jnp.dot(sfv, g10_ref[...], preferred_element_type=jnp.float32)
             + jnp.dot(neigh, g11_ref[...],
                       preferred_element_type=jnp.float32)
             + jnp.dot(sfv * neigh, g12_ref[...],
                       preferred_element_type=jnp.float32)
             + g1b_ref[...])
        beta = jax.nn.sigmoid(z)
        out_ref[...] = beta * sfv + (1.0 - beta) * neigh

    return pl.pallas_call(
        kern,
        out_shape=jax.ShapeDtypeStruct((BP, DP), jnp.float32),
    )(embed_p, sf_p, inwb, inbp, outwb, outbp, g10, g11, g12, g1bp,
      bngp, bnbp, bn1gp, bn1bp, mavg)


def _bd(w):
    return jnp.kron(jnp.eye(PK, dtype=jnp.float32), w)


def _tile_row(v):
    return jnp.tile(v, PK).reshape(1, DP)


def kernel(nodes, history_ui, history_r, u2e_w, i2e_w, r2e_w, l1W, l1b,
           a1W, a1b, a2W, a2b, a3W, a3b, gate_W, gate_b, gate1_W, gate1_b,
           bn_g, bn_b, inW, inb, bn1_g, bn1_b, outW, outb):
    hist_idx = history_ui.astype(jnp.int32).T.reshape(-1)
    nodes_idx = nodes.astype(jnp.int32)

    i2e_lin = _tc_linearize(i2e_w).reshape(i2e_w.shape)
    u2e_lin = _tc_linearize(u2e_w).reshape(u2e_w.shape)

    eui_flat, sf = _sc_gather(hist_idx, nodes_idx, i2e_lin, u2e_lin)
    eui_p = eui_flat.reshape(L, BP, DP)
    sf_p = sf.reshape(BP, DP)

    hr3 = history_r.astype(jnp.int32).T.reshape(L, BP, PK)
    hr3 = jnp.transpose(hr3, (0, 2, 1))
    kk = jnp.arange(PK * NR5, dtype=jnp.int32)
    oh_p = (hr3[:, kk // NR5, :] == (kk % NR5)[None, :, None]
            ).astype(jnp.float32)

    gate_wt = gate_W.T
    embed_p = _tc_stage1(
        eui_p, oh_p,
        _bd(r2e_w),
        _bd(gate_wt[:D]), _bd(gate_wt[D:2 * D]), _bd(gate_wt[2 * D:]),
        jnp.kron(jnp.eye(PK, dtype=jnp.float32),
                 jnp.ones((D, D), jnp.float32)),
        _tile_row(gate_b))

    g1t = gate1_W.T
    mavg = jnp.kron(jnp.full((PK, PK), 1.0 / PK, jnp.float32),
                    jnp.eye(D, dtype=jnp.float32))
    out_p = _tc_stage2(
        embed_p, sf_p,
        _bd(inW.T), _tile_row(inb), _bd(outW.T), _tile_row(outb),
        _bd(g1t[:D]), _bd(g1t[D:2 * D]), _bd(g1t[2 * D:]), _tile_row(gate1_b),
        _tile_row(bn_g), _tile_row(bn_b), _tile_row(bn1_g), _tile_row(bn1_b),
        mavg)

    return out_p.reshape(B, D)


# speedup vs baseline: 6.1746x; 1.0405x over previous
import functools

import jax
import jax.numpy as jnp
from jax import lax
from jax.experimental import pallas as pl
from jax.experimental.pallas import tpu as pltpu
from jax.experimental.pallas import tpu_sc as plsc

B = 4096
L = 50
D = 32
NR5 = 5
EPS_BN = 1e-5

NW = 32
RTOT = B * L
RPW = RTOT // NW
CH = 128
NCH = RPW // CH
BPW = B // NW

PK = 4
BP = B // PK
DP = PK * D


def _sc_gather(hist_idx, nodes_idx, i2e_w, u2e_w):
    mesh = plsc.VectorSubcoreMesh(core_axis_name="c", subcore_axis_name="s")

    @functools.partial(
        pl.kernel,
        mesh=mesh,
        compiler_params=pltpu.CompilerParams(use_tc_tiling_on_sc=False),
        out_type=(
            jax.ShapeDtypeStruct((RTOT, D), jnp.float32),
            jax.ShapeDtypeStruct((B, D), jnp.float32),
        ),
        scratch_types=[
            pltpu.VMEM((NCH, CH), jnp.int32),
            pltpu.VMEM((CH, D), jnp.float32),
            pltpu.VMEM((CH, D), jnp.float32),
            pltpu.VMEM((1, CH), jnp.int32),
            pltpu.VMEM((CH, D), jnp.float32),
            pltpu.SemaphoreType.DMA,
        ],
    )
    def k(idx_hbm, nodes_hbm, i2e_hbm, u2e_hbm, eui_out, self_out,
          idxv, buf0, buf1, idxu, bufu, sem):
        c = lax.axis_index("c")
        s = lax.axis_index("s")
        wid = s * 2 + c
        pltpu.sync_copy(idx_hbm.at[wid], idxv)

        def body(j2, _):
            j = j2 * 2
            base0 = pl.multiple_of(wid * RPW + j * CH, CH)
            base1 = pl.multiple_of(wid * RPW + (j + 1) * CH, CH)
            h0 = pltpu.async_copy(i2e_hbm.at[idxv.at[j]], buf0, sem)
            h1 = pltpu.async_copy(i2e_hbm.at[idxv.at[j + 1]], buf1, sem)
            h0.wait()
            pltpu.sync_copy(buf0, eui_out.at[pl.ds(base0, CH)])
            h1.wait()
            pltpu.sync_copy(buf1, eui_out.at[pl.ds(base1, CH)])
            return 0

        lax.fori_loop(0, NCH // 2, body, 0, unroll=False)

        pltpu.sync_copy(nodes_hbm.at[wid], idxu)
        pltpu.async_copy(u2e_hbm.at[idxu.at[0]], bufu, sem).wait()
        sbase = pl.multiple_of(wid * BPW, BPW)
        pltpu.sync_copy(bufu, self_out.at[pl.ds(sbase, BPW)])

    return k(hist_idx.reshape(NW, NCH, CH), nodes_idx.reshape(NW, 1, BPW),
             i2e_w, u2e_w)


TK = 4096


def _tc_linearize(table):
    n = table.shape[0]
    nblk = -(-n // TK)
    tq = TK // PK

    def kern(x_ref, o_ref):
        for q in range(PK):
            o_ref[:, q * D:(q + 1) * D] = jnp.transpose(
                x_ref[:, q * tq:(q + 1) * tq])

    return pl.pallas_call(
        kern,
        grid=(nblk,),
        in_specs=[pl.BlockSpec((D, TK), lambda i: (0, i))],
        out_specs=pl.BlockSpec((tq, DP), lambda i: (i, 0)),
        out_shape=jax.ShapeDtypeStruct((nblk * tq, DP), jnp.float32),
    )(table.T)


def _perm_idx(r):
    return ((r >> 12) << 12) | ((r & 1023) << 2) | ((r >> 10) & 3)


NBP = 256


def _tc_stage1(eui_p, oh_p, w20, g0b, g1b, g2b, bd1, gbp):
    def kern(eui_ref, oh_ref, w20_ref, g0_ref, g1_ref, g2_ref, bd1_ref,
             gb_ref, out_ref):
        w20v = w20_ref[...]
        g0 = g0_ref[...]
        g1 = g1_ref[...]
        g2 = g2_ref[...]
        bd1 = bd1_ref[...]
        gb = gb_ref[...]
        acc = jnp.zeros((NBP, DP), jnp.float32)
        for j in range(L):
            e = eui_ref[j]
            r = lax.dot_general(oh_ref[j], w20v, (((0,), (0,)), ((), ())),
                                preferred_element_type=jnp.float32)
            x = e * r
            z = (jnp.dot(e, g0, preferred_element_type=jnp.float32)
                 + jnp.dot(r, g1, preferred_element_type=jnp.float32)
                 + jnp.dot(x, g2, preferred_element_type=jnp.float32)
                 + gb)
            alpha = jax.nn.sigmoid(z)
            o = alpha * e + (1.0 - alpha) * r
            n2 = jnp.dot(o * o, bd1, preferred_element_type=jnp.float32)
            n = jnp.sqrt(n2)
            acc = acc + o / jnp.maximum(n, 1e-12)
        out_ref[...] = acc

    return pl.pallas_call(
        kern,
        grid=(BP // NBP,),
        in_specs=[
            pl.BlockSpec((L, NBP, DP), lambda b: (0, b, 0)),
            pl.BlockSpec((L, PK * NR5, NBP), lambda b: (0, 0, b)),
            pl.BlockSpec((PK * NR5, DP), lambda b: (0, 0)),
            pl.BlockSpec((DP, DP), lambda b: (0, 0)),
            pl.BlockSpec((DP, DP), lambda b: (0, 0)),
            pl.BlockSpec((DP, DP), lambda b: (0, 0)),
            pl.BlockSpec((DP, DP), lambda b: (0, 0)),
            pl.BlockSpec((1, DP), lambda b: (0, 0)),
        ],
        out_specs=pl.BlockSpec((NBP, DP), lambda b: (b, 0)),
        out_shape=jax.ShapeDtypeStruct((BP, DP), jnp.float32),
    )(eui_p, oh_p, w20, g0b, g1b, g2b, bd1, gbp)


def _selu(x):
    a = 1.6732632423543772848170429916717
    s = 1.0507009873554804934193349852946
    return s * jnp.where(x > 0, x, a * (jnp.exp(x) - 1.0))


def _tc_stage2(embed_p, sf_p, inwb, inbp, outwb, outbp, g10, g11, g12, g1bp,
               bngp, bnbp, bn1gp, bn1bp, mavg):
    def kern(em_ref, sf_ref, inw_ref, inb_ref, outw_ref, outb_ref,
             g10_ref, g11_ref, g12_ref, g1b_ref, bng_ref, bnb_ref,
             bn1g_ref, bn1b_ref, mavg_ref, out_ref):
        em = em_ref[...]
        mv = mavg_ref[...]
        m = jnp.dot(jnp.mean(em, axis=0, keepdims=True), mv,
                    preferred_element_type=jnp.float32)
        v = jnp.dot(jnp.mean((em - m) ** 2, axis=0, keepdims=True), mv,
                    preferred_element_type=jnp.float32)
        xb = (em - m) / jnp.sqrt(v + EPS_BN) * bng_ref[...] + bnb_ref[...]
        xb = _selu(jnp.dot(xb, inw_ref[...],
                           preferred_element_type=jnp.float32) + inb_ref[...])
        m1 = jnp.dot(jnp.mean(xb, axis=0, keepdims=True), mv,
                     preferred_element_type=jnp.float32)
        v1 = jnp.dot(jnp.mean((xb - m1) ** 2, axis=0, keepdims=True), mv,
                     preferred_element_type=jnp.float32)
        xb = (xb - m1) / jnp.sqrt(v1 + EPS_BN) * bn1g_ref[...] + bn1b_ref[...]
        neigh = jnp.dot(xb, outw_ref[...],
                        preferred_element_type=jnp.float32) + outb_ref[...]
        sfv = sf_ref[...]
        z = (jnp.dot(sfv, g10_ref[...], preferred_element_type=jnp.float32)
             + jnp.dot(neigh, g11_ref[...],
                       preferred_element_type=jnp.float32)
             + jnp.dot(sfv * neigh, g12_ref[...],
                       preferred_element_type=jnp.float32)
             + g1b_ref[...])
        beta = jax.nn.sigmoid(z)
        out_ref[...] = beta * sfv + (1.0 - beta) * neigh

    return pl.pallas_call(
        kern,
        out_shape=jax.ShapeDtypeStruct((BP, DP), jnp.float32),
    )(embed_p, sf_p, inwb, inbp, outwb, outbp, g10, g11, g12, g1bp,
      bngp, bnbp, bn1gp, bn1bp, mavg)


def _bd(w):
    return jnp.kron(jnp.eye(PK, dtype=jnp.float32), w)


def _tile_row(v):
    return jnp.tile(v, PK).reshape(1, DP)


def kernel(nodes, history_ui, history_r, u2e_w, i2e_w, r2e_w, l1W, l1b,
           a1W, a1b, a2W, a2b, a3W, a3b, gate_W, gate_b, gate1_W, gate1_b,
           bn_g, bn_b, inW, inb, bn1_g, bn1_b, outW, outb):
    hist_idx = _perm_idx(history_ui.astype(jnp.int32)).T.reshape(-1)
    nodes_idx = _perm_idx(nodes.astype(jnp.int32))

    i2e_lin = _tc_linearize(i2e_w).reshape(-1, D)
    u2e_lin = _tc_linearize(u2e_w).reshape(-1, D)

    eui_flat, sf = _sc_gather(hist_idx, nodes_idx, i2e_lin, u2e_lin)
    eui_p = eui_flat.reshape(L, BP, DP)
    sf_p = sf.reshape(BP, DP)

    hr3 = history_r.astype(jnp.int32).T.reshape(L, BP, PK)
    hr3 = jnp.transpose(hr3, (0, 2, 1))
    kk = jnp.arange(PK * NR5, dtype=jnp.int32)
    oh_p = (hr3[:, kk // NR5, :] == (kk % NR5)[None, :, None]
            ).astype(jnp.float32)

    gate_wt = gate_W.T
    embed_p = _tc_stage1(
        eui_p, oh_p,
        _bd(r2e_w),
        _bd(gate_wt[:D]), _bd(gate_wt[D:2 * D]), _bd(gate_wt[2 * D:]),
        jnp.kron(jnp.eye(PK, dtype=jnp.float32),
                 jnp.ones((D, D), jnp.float32)),
        _tile_row(gate_b))

    g1t = gate1_W.T
    mavg = jnp.kron(jnp.full((PK, PK), 1.0 / PK, jnp.float32),
                    jnp.eye(D, dtype=jnp.float32))
    out_p = _tc_stage2(
        embed_p, sf_p,
        _bd(inW.T), _tile_row(inb), _bd(outW.T), _tile_row(outb),
        _bd(g1t[:D]), _bd(g1t[D:2 * D]), _bd(g1t[2 * D:]), _tile_row(gate1_b),
        _tile_row(bn_g), _tile_row(bn_b), _tile_row(bn1_g), _tile_row(bn1_b),
        mavg)

    return out_p.reshape(B, D)


# speedup vs baseline: 6.8605x; 1.1111x over previous
import functools

import jax
import jax.numpy as jnp
from jax import lax
from jax.experimental import pallas as pl
from jax.experimental.pallas import tpu as pltpu
from jax.experimental.pallas import tpu_sc as plsc

B = 4096
L = 50
D = 32
NR5 = 5
EPS_BN = 1e-5

NW = 32
RTOT = B * L
RPW = RTOT // NW
CH = 128
NCH = RPW // CH
BPW = B // NW

PK = 4
BP = B // PK
DP = PK * D


def _sc_gather(hist_idx, nodes_idx, i2e_w, u2e_w):
    mesh = plsc.VectorSubcoreMesh(core_axis_name="c", subcore_axis_name="s")

    @functools.partial(
        pl.kernel,
        mesh=mesh,
        compiler_params=pltpu.CompilerParams(use_tc_tiling_on_sc=False),
        out_type=(
            jax.ShapeDtypeStruct((RTOT, D), jnp.float32),
            jax.ShapeDtypeStruct((B, D), jnp.float32),
        ),
        scratch_types=[
            pltpu.VMEM((NCH, CH), jnp.int32),
            pltpu.VMEM((CH, D), jnp.float32),
            pltpu.VMEM((CH, D), jnp.float32),
            pltpu.VMEM((1, CH), jnp.int32),
            pltpu.VMEM((CH, D), jnp.float32),
            pltpu.SemaphoreType.DMA,
        ],
    )
    def k(idx_hbm, nodes_hbm, i2e_hbm, u2e_hbm, eui_out, self_out,
          idxv, buf0, buf1, idxu, bufu, sem):
        c = lax.axis_index("c")
        s = lax.axis_index("s")
        wid = s * 2 + c
        pltpu.sync_copy(idx_hbm.at[wid], idxv)

        def body(j2, _):
            j = j2 * 2
            base0 = pl.multiple_of(wid * RPW + j * CH, CH)
            base1 = pl.multiple_of(wid * RPW + (j + 1) * CH, CH)
            h0 = pltpu.async_copy(i2e_hbm.at[idxv.at[j]], buf0, sem)
            h1 = pltpu.async_copy(i2e_hbm.at[idxv.at[j + 1]], buf1, sem)
            h0.wait()
            pltpu.sync_copy(buf0, eui_out.at[pl.ds(base0, CH)])
            h1.wait()
            pltpu.sync_copy(buf1, eui_out.at[pl.ds(base1, CH)])
            return 0

        lax.fori_loop(0, NCH // 2, body, 0, unroll=False)

        pltpu.sync_copy(nodes_hbm.at[wid], idxu)
        pltpu.async_copy(u2e_hbm.at[idxu.at[0]], bufu, sem).wait()
        sbase = pl.multiple_of(wid * BPW, BPW)
        pltpu.sync_copy(bufu, self_out.at[pl.ds(sbase, BPW)])

    return k(hist_idx.reshape(NW, NCH, CH), nodes_idx.reshape(NW, 1, BPW),
             i2e_w, u2e_w)


TK = 8192
TKQ = TK // PK


def _tc_linearize(table):
    n = table.shape[0]
    nblk = -(-n // TK)
    tq = TK // PK

    def kern(x_ref, o_ref):
        for q in range(PK):
            o_ref[:, q * D:(q + 1) * D] = jnp.transpose(
                x_ref[:, q * tq:(q + 1) * tq])

    return pl.pallas_call(
        kern,
        grid=(nblk,),
        in_specs=[pl.BlockSpec((D, TK), lambda i: (0, i))],
        out_specs=pl.BlockSpec((tq, DP), lambda i: (i, 0)),
        out_shape=jax.ShapeDtypeStruct((nblk * tq, DP), jnp.float32),
    )(table.T)


_SH_BLK = TK.bit_length() - 1
_SH_Q = TKQ.bit_length() - 1


def _perm_idx(r):
    return (((r >> _SH_BLK) << _SH_BLK) | ((r & (TKQ - 1)) << 2)
            | ((r >> _SH_Q) & 3))


NBP = 256


def _tc_stage1(eui_p, oh_p, w20, g0b, g1b, g2b, bd1, gbp):
    def kern(eui_ref, oh_ref, w20_ref, g0_ref, g1_ref, g2_ref, bd1_ref,
             gb_ref, out_ref):
        w20v = w20_ref[...]
        g0 = g0_ref[...]
        g1 = g1_ref[...]
        g2 = g2_ref[...]
        bd1 = bd1_ref[...]
        gb = gb_ref[...]
        acc = jnp.zeros((NBP, DP), jnp.float32)
        for j in range(L):
            e = eui_ref[j]
            r = lax.dot_general(oh_ref[j], w20v, (((0,), (0,)), ((), ())),
                                preferred_element_type=jnp.float32)
            x = e * r
            z = (jnp.dot(e, g0, preferred_element_type=jnp.float32)
                 + jnp.dot(r, g1, preferred_element_type=jnp.float32)
                 + jnp.dot(x, g2, preferred_element_type=jnp.float32)
                 + gb)
            alpha = jax.nn.sigmoid(z)
            o = alpha * e + (1.0 - alpha) * r
            n2 = jnp.dot(o * o, bd1, preferred_element_type=jnp.float32)
            n = jnp.sqrt(n2)
            acc = acc + o / jnp.maximum(n, 1e-12)
        out_ref[...] = acc

    return pl.pallas_call(
        kern,
        grid=(BP // NBP,),
        in_specs=[
            pl.BlockSpec((L, NBP, DP), lambda b: (0, b, 0)),
            pl.BlockSpec((L, PK * NR5, NBP), lambda b: (0, 0, b)),
            pl.BlockSpec((PK * NR5, DP), lambda b: (0, 0)),
            pl.BlockSpec((DP, DP), lambda b: (0, 0)),
            pl.BlockSpec((DP, DP), lambda b: (0, 0)),
            pl.BlockSpec((DP, DP), lambda b: (0, 0)),
            pl.BlockSpec((DP, DP), lambda b: (0, 0)),
            pl.BlockSpec((1, DP), lambda b: (0, 0)),
        ],
        out_specs=pl.BlockSpec((NBP, DP), lambda b: (b, 0)),
        out_shape=jax.ShapeDtypeStruct((BP, DP), jnp.float32),
    )(eui_p, oh_p, w20, g0b, g1b, g2b, bd1, gbp)


def _selu(x):
    a = 1.6732632423543772848170429916717
    s = 1.0507009873554804934193349852946
    return s * jnp.where(x > 0, x, a * (jnp.exp(x) - 1.0))


def _tc_stage2(embed_p, sf_p, inwb, inbp, outwb, outbp, g10, g11, g12, g1bp,
               bngp, bnbp, bn1gp, bn1bp, mavg):
    def kern(em_ref, sf_ref, inw_ref, inb_ref, outw_ref, outb_ref,
             g10_ref, g11_ref, g12_ref, g1b_ref, bng_ref, bnb_ref,
             bn1g_ref, bn1b_ref, mavg_ref, out_ref):
        em = em_ref[...]
        mv = mavg_ref[...]
        m = jnp.dot(jnp.mean(em, axis=0, keepdims=True), mv,
                    preferred_element_type=jnp.float32)
        v = jnp.dot(jnp.mean((em - m) ** 2, axis=0, keepdims=True), mv,
                    preferred_element_type=jnp.float32)
        xb = (em - m) / jnp.sqrt(v + EPS_BN) * bng_ref[...] + bnb_ref[...]
        xb = _selu(jnp.dot(xb, inw_ref[...],
                           preferred_element_type=jnp.float32) + inb_ref[...])
        m1 = jnp.dot(jnp.mean(xb, axis=0, keepdims=True), mv,
                     preferred_element_type=jnp.float32)
        v1 = jnp.dot(jnp.mean((xb - m1) ** 2, axis=0, keepdims=True), mv,
                     preferred_element_type=jnp.float32)
        xb = (xb - m1) / jnp.sqrt(v1 + EPS_BN) * bn1g_ref[...] + bn1b_ref[...]
        neigh = jnp.dot(xb, outw_ref[...],
                        preferred_element_type=jnp.float32) + outb_ref[...]
        sfv = sf_ref[...]
        z = (jnp.dot(sfv, g10_ref[...], preferred_element_type=jnp.float32)
             + jnp.dot(neigh, g11_ref[...],
                       preferred_element_type=jnp.float32)
             + jnp.dot(sfv * neigh, g12_ref[...],
                       preferred_element_type=jnp.float32)
             + g1b_ref[...])
        beta = jax.nn.sigmoid(z)
        out_ref[...] = beta * sfv + (1.0 - beta) * neigh

    return pl.pallas_call(
        kern,
        out_shape=jax.ShapeDtypeStruct((BP, DP), jnp.float32),
    )(embed_p, sf_p, inwb, inbp, outwb, outbp, g10, g11, g12, g1bp,
      bngp, bnbp, bn1gp, bn1bp, mavg)


def _bd(w):
    return jnp.kron(jnp.eye(PK, dtype=jnp.float32), w)


def _tile_row(v):
    return jnp.tile(v, PK).reshape(1, DP)


def kernel(nodes, history_ui, history_r, u2e_w, i2e_w, r2e_w, l1W, l1b,
           a1W, a1b, a2W, a2b, a3W, a3b, gate_W, gate_b, gate1_W, gate1_b,
           bn_g, bn_b, inW, inb, bn1_g, bn1_b, outW, outb):
    hist_idx = _perm_idx(history_ui.astype(jnp.int32)).T.reshape(-1)
    nodes_idx = _perm_idx(nodes.astype(jnp.int32))

    i2e_lin = _tc_linearize(i2e_w).reshape(-1, D)
    u2e_lin = _tc_linearize(u2e_w).reshape(-1, D)

    eui_flat, sf = _sc_gather(hist_idx, nodes_idx, i2e_lin, u2e_lin)
    eui_p = eui_flat.reshape(L, BP, DP)
    sf_p = sf.reshape(BP, DP)

    hr3 = history_r.astype(jnp.int32).T.reshape(L, BP, PK)
    hr3 = jnp.transpose(hr3, (0, 2, 1))
    kk = jnp.arange(PK * NR5, dtype=jnp.int32)
    oh_p = (hr3[:, kk // NR5, :] == (kk % NR5)[None, :, None]
            ).astype(jnp.float32)

    gate_wt = gate_W.T
    embed_p = _tc_stage1(
        eui_p, oh_p,
        _bd(r2e_w),
        _bd(gate_wt[:D]), _bd(gate_wt[D:2 * D]), _bd(gate_wt[2 * D:]),
        jnp.kron(jnp.eye(PK, dtype=jnp.float32),
                 jnp.ones((D, D), jnp.float32)),
        _tile_row(gate_b))

    g1t = gate1_W.T
    mavg = jnp.kron(jnp.full((PK, PK), 1.0 / PK, jnp.float32),
                    jnp.eye(D, dtype=jnp.float32))
    out_p = _tc_stage2(
        embed_p, sf_p,
        _bd(inW.T), _tile_row(inb), _bd(outW.T), _tile_row(outb),
        _bd(g1t[:D]), _bd(g1t[D:2 * D]), _bd(g1t[2 * D:]), _tile_row(gate1_b),
        _tile_row(bn_g), _tile_row(bn_b), _tile_row(bn1_g), _tile_row(bn1_b),
        mavg)

    return out_p.reshape(B, D)
